# Initial kernel scaffold; baseline (speedup 1.0000x reference)
import jax
import jax.numpy as jnp
from jax.experimental import pallas as pl


def kernel(pred_x, pred_q, target_x, target_q, edge2graph, node2graph, atom_type, edge_r, edge_p):
    raise NotImplementedError("write your pallas kernel here")


# speedup vs baseline: 26.6635x; 26.6635x over previous
import jax
import jax.numpy as jnp
from jax import lax
from jax.experimental import pallas as pl
from jax.experimental.pallas import tpu as pltpu
from jax.experimental.pallas import tpu_sc as plsc

NUM_SEG = 512
SEG_PAD = 513
NE = 3_200_000
NN = 100_000
NCOMP = 3 * NN

NC, NS, L = 2, 16, 16
NW = NC * NS

E_PER_W = NE // NW
CE = 4_000
N_CHUNKS = E_PER_W // CE

NPW = 9_376
NPW_LAST = NCOMP - (NW - 1) * NPW


def _sc_body(pq_hbm, tq_hbm, eg_hbm, px_hbm, tx_hbm, ng_hbm, out_hbm,
             bpq, btq, bsg, npx, ntx, nng, tab, sem_n):
    wid = lax.axis_index("s") * NC + lax.axis_index("c")
    lane = lax.iota(jnp.int32, L)
    zero = jnp.zeros((L,), jnp.float32)
    ones = jnp.ones((L,), jnp.float32)

    def q_idx(q):
        return jnp.full((L,), q, jnp.int32)

    nbase = jnp.minimum(wid * NPW, NCOMP - NPW)
    h_px = pltpu.async_copy(px_hbm.at[pl.ds(nbase, NPW)], npx, sem_n)
    h_tx = pltpu.async_copy(tx_hbm.at[pl.ds(nbase, NPW)], ntx, sem_n)
    h_ng = pltpu.async_copy(ng_hbm.at[pl.ds(nbase, NPW)], nng, sem_n)

    def zbody(r, _):
        for q in range(7):
            for l in range(L):
                tab[q, l, pl.ds(r * L, L)] = zero
        return 0

    lax.fori_loop(0, SEG_PAD // L, zbody, 0)
    for q in range(7):
        for l in range(L):
            tab[q, l, pl.ds(SEG_PAD - L, L)] = zero

    ebase = wid * E_PER_W

    def ebody(i, _):
        o = i * L
        pq = bpq[pl.ds(o, L)]
        tq = btq[pl.ds(o, L)]
        sg = bsg[pl.ds(o, L)]
        d = pq - tq
        plsc.addupdate_scatter(tab, [q_idx(0), lane, sg], d * d)
        plsc.addupdate_scatter(tab, [q_idx(1), lane, sg], tq * tq)
        plsc.addupdate_scatter(tab, [q_idx(2), lane, sg], pq * pq)
        return 0

    def echunk(k, _):
        off = ebase + k * CE
        pltpu.sync_copy(pq_hbm.at[pl.ds(off, CE)], bpq)
        pltpu.sync_copy(tq_hbm.at[pl.ds(off, CE)], btq)
        pltpu.sync_copy(eg_hbm.at[pl.ds(off, CE)], bsg)
        lax.fori_loop(0, CE // L, ebody, 0)
        return 0

    lax.fori_loop(0, N_CHUNKS, echunk, 0)

    h_px.wait()
    h_tx.wait()
    h_ng.wait()

    is_last = wid == NW - 1
    so = jnp.where(is_last, NPW - NPW_LAST, 0)
    n_it = jnp.where(is_last, NPW_LAST // L, NPW // L)

    def nbody(i, _):
        o = so + i * L
        px = npx[pl.ds(o, L)]
        tx = ntx[pl.ds(o, L)]
        sg = nng[pl.ds(o, L)]
        d = px - tx
        plsc.addupdate_scatter(tab, [q_idx(3), lane, sg], d * d)
        plsc.addupdate_scatter(tab, [q_idx(4), lane, sg], tx * tx)
        plsc.addupdate_scatter(tab, [q_idx(5), lane, sg], px * px)
        plsc.addupdate_scatter(tab, [q_idx(6), lane, sg], ones)
        return 0

    lax.fori_loop(0, n_it, nbody, 0)

    pltpu.sync_copy(tab, out_hbm.at[wid])


def _sc_accumulate(pq, tq, eg, px, tx, ng):
    mesh = plsc.VectorSubcoreMesh(
        core_axis_name="c", subcore_axis_name="s", num_cores=NC, num_subcores=NS
    )
    f = pl.kernel(
        _sc_body,
        out_type=jax.ShapeDtypeStruct((NW, 7, L, SEG_PAD), jnp.float32),
        mesh=mesh,
        scratch_types=[
            pltpu.VMEM((CE,), jnp.float32),
            pltpu.VMEM((CE,), jnp.float32),
            pltpu.VMEM((CE,), jnp.int32),
            pltpu.VMEM((NPW,), jnp.float32),
            pltpu.VMEM((NPW,), jnp.float32),
            pltpu.VMEM((NPW,), jnp.int32),
            pltpu.VMEM((7, L, SEG_PAD), jnp.float32),
            pltpu.SemaphoreType.DMA,
        ],
        compiler_params=pltpu.CompilerParams(
            use_tc_tiling_on_sc=False, needs_layout_passes=False
        ),
    )
    return f(pq, tq, eg, px, tx, ng)


def _epi_body(t_ref, o_ref):
    t = t_ref[...]
    s = jnp.sum(t, axis=(0, 2))[:, :NUM_SEG]
    cnt = s[6:7] / 3.0
    nerr = jnp.sqrt(s[0:1])
    denq = jnp.sqrt(s[1:2])
    psq = jnp.sqrt(s[2:3])
    perrq = nerr / denq
    rmsd = jnp.sqrt(s[3:4] / cnt)
    denx = jnp.sqrt(s[4:5] / cnt)
    psx = jnp.sqrt(s[5:6] / cnt)
    perrx = rmsd / denx
    out8 = jnp.concatenate(
        [rmsd, perrx, psx, denx, nerr, perrq, psq, denq], axis=0
    )
    o_ref[...] = jnp.sum(out8, axis=-1)


def _tc_epilogue(part):
    return pl.pallas_call(
        _epi_body,
        out_shape=jax.ShapeDtypeStruct((8,), jnp.float32),
    )(part)


def kernel(pred_x, pred_q, target_x, target_q, edge2graph, node2graph,
           atom_type, edge_r, edge_p):
    del atom_type, edge_r, edge_p
    pxf = pred_x.reshape(-1)
    txf = target_x.reshape(-1)
    c2g = jnp.repeat(node2graph.astype(jnp.int32), 3)
    eg = edge2graph.astype(jnp.int32)
    part = _sc_accumulate(pred_q, target_q, eg, pxf, txf, c2g)
    return _tc_epilogue(part)


# speedup vs baseline: 30.9805x; 1.1619x over previous
import jax
import jax.numpy as jnp
from jax import lax
from jax.experimental import pallas as pl
from jax.experimental.pallas import tpu as pltpu
from jax.experimental.pallas import tpu_sc as plsc

NUM_SEG = 512
SEG_PAD = 513
NE = 3_200_000
NN = 100_000
NCOMP = 3 * NN

NC, NS, L = 2, 16, 16
NW = NC * NS

E_PER_W = NE // NW
CE = 4_000
N_CHUNKS = E_PER_W // CE

NPW = 9_376
NPW_LAST = NCOMP - (NW - 1) * NPW


def _sc_body(pq_hbm, tq_hbm, eg_hbm, px_hbm, tx_hbm, ng_hbm, out_hbm,
             bpq, btq, bsg, npx, ntx, nng, tab, sem_n, sem_e0, sem_e1):
    wid = lax.axis_index("s") * NC + lax.axis_index("c")
    lane = lax.iota(jnp.int32, L)
    zero = jnp.zeros((L,), jnp.float32)
    ones = jnp.ones((L,), jnp.float32)

    def q_idx(q):
        return jnp.full((L,), q, jnp.int32)

    nbase = jnp.minimum(wid * NPW, NCOMP - NPW)
    h_px = pltpu.async_copy(px_hbm.at[pl.ds(nbase, NPW)], npx, sem_n)
    h_tx = pltpu.async_copy(tx_hbm.at[pl.ds(nbase, NPW)], ntx, sem_n)
    h_ng = pltpu.async_copy(ng_hbm.at[pl.ds(nbase, NPW)], nng, sem_n)

    def zbody(r, _):
        for q in range(7):
            for l in range(L):
                tab[q, l, pl.ds(r * L, L)] = zero
        return 0

    lax.fori_loop(0, SEG_PAD // L, zbody, 0)
    for q in range(7):
        for l in range(L):
            tab[q, l, pl.ds(SEG_PAD - L, L)] = zero

    ebase = wid * E_PER_W
    esems = (sem_e0, sem_e1)

    def e_start(k, b):
        off = ebase + k * CE
        pltpu.async_copy(pq_hbm.at[pl.ds(off, CE)], bpq.at[b], esems[b])
        pltpu.async_copy(tq_hbm.at[pl.ds(off, CE)], btq.at[b], esems[b])
        pltpu.async_copy(eg_hbm.at[pl.ds(off, CE)], bsg.at[b], esems[b])

    def e_wait(b):
        pltpu.make_async_copy(pq_hbm.at[pl.ds(0, CE)], bpq.at[b], esems[b]).wait()
        pltpu.make_async_copy(tq_hbm.at[pl.ds(0, CE)], btq.at[b], esems[b]).wait()
        pltpu.make_async_copy(eg_hbm.at[pl.ds(0, CE)], bsg.at[b], esems[b]).wait()

    def e_compute(b):
        def ebody(i, _):
            o = i * L
            pq = bpq[b, pl.ds(o, L)]
            tq = btq[b, pl.ds(o, L)]
            sg = bsg[b, pl.ds(o, L)]
            d = pq - tq
            plsc.addupdate_scatter(tab, [q_idx(0), lane, sg], d * d)
            plsc.addupdate_scatter(tab, [q_idx(1), lane, sg], tq * tq)
            plsc.addupdate_scatter(tab, [q_idx(2), lane, sg], pq * pq)
            return 0

        lax.fori_loop(0, CE // L, ebody, 0)

    e_start(0, 0)
    e_start(1, 1)

    def pair(j, _):
        k0 = 2 * j

        e_wait(0)

        @pl.when(k0 + 2 < N_CHUNKS)
        def _s0():
            e_start(k0 + 2, 0)

        e_compute(0)

        e_wait(1)

        @pl.when(k0 + 3 < N_CHUNKS)
        def _s1():
            e_start(k0 + 3, 1)

        e_compute(1)
        return 0

    lax.fori_loop(0, N_CHUNKS // 2, pair, 0)
    if N_CHUNKS % 2:
        e_wait(0)
        e_compute(0)

    h_px.wait()
    h_tx.wait()
    h_ng.wait()

    is_last = wid == NW - 1
    so = jnp.where(is_last, NPW - NPW_LAST, 0)
    n_it = jnp.where(is_last, NPW_LAST // L, NPW // L)

    def nbody(i, _):
        o = so + i * L
        px = npx[pl.ds(o, L)]
        tx = ntx[pl.ds(o, L)]
        sg = nng[pl.ds(o, L)]
        d = px - tx
        plsc.addupdate_scatter(tab, [q_idx(3), lane, sg], d * d)
        plsc.addupdate_scatter(tab, [q_idx(4), lane, sg], tx * tx)
        plsc.addupdate_scatter(tab, [q_idx(5), lane, sg], px * px)
        plsc.addupdate_scatter(tab, [q_idx(6), lane, sg], ones)
        return 0

    lax.fori_loop(0, n_it, nbody, 0)

    pltpu.sync_copy(tab, out_hbm.at[wid])


def _sc_accumulate(pq, tq, eg, px, tx, ng):
    mesh = plsc.VectorSubcoreMesh(
        core_axis_name="c", subcore_axis_name="s", num_cores=NC, num_subcores=NS
    )
    f = pl.kernel(
        _sc_body,
        out_type=jax.ShapeDtypeStruct((NW, 7, L, SEG_PAD), jnp.float32),
        mesh=mesh,
        scratch_types=[
            pltpu.VMEM((2, CE), jnp.float32),
            pltpu.VMEM((2, CE), jnp.float32),
            pltpu.VMEM((2, CE), jnp.int32),
            pltpu.VMEM((NPW,), jnp.float32),
            pltpu.VMEM((NPW,), jnp.float32),
            pltpu.VMEM((NPW,), jnp.int32),
            pltpu.VMEM((7, L, SEG_PAD), jnp.float32),
            pltpu.SemaphoreType.DMA,
            pltpu.SemaphoreType.DMA,
            pltpu.SemaphoreType.DMA,
        ],
        compiler_params=pltpu.CompilerParams(
            use_tc_tiling_on_sc=False, needs_layout_passes=False
        ),
    )
    return f(pq, tq, eg, px, tx, ng)


def _epi_body(t_ref, o_ref):
    t = t_ref[...]
    s = jnp.sum(t, axis=(0, 2))[:, :NUM_SEG]
    cnt = s[6:7] / 3.0
    nerr = jnp.sqrt(s[0:1])
    denq = jnp.sqrt(s[1:2])
    psq = jnp.sqrt(s[2:3])
    perrq = nerr / denq
    rmsd = jnp.sqrt(s[3:4] / cnt)
    denx = jnp.sqrt(s[4:5] / cnt)
    psx = jnp.sqrt(s[5:6] / cnt)
    perrx = rmsd / denx
    out8 = jnp.concatenate(
        [rmsd, perrx, psx, denx, nerr, perrq, psq, denq], axis=0
    )
    o_ref[...] = jnp.sum(out8, axis=-1)


def _tc_epilogue(part):
    return pl.pallas_call(
        _epi_body,
        out_shape=jax.ShapeDtypeStruct((8,), jnp.float32),
    )(part)


def kernel(pred_x, pred_q, target_x, target_q, edge2graph, node2graph,
           atom_type, edge_r, edge_p):
    del atom_type, edge_r, edge_p
    pxf = pred_x.reshape(-1)
    txf = target_x.reshape(-1)
    c2g = jnp.repeat(node2graph.astype(jnp.int32), 3)
    eg = edge2graph.astype(jnp.int32)
    part = _sc_accumulate(pred_q, target_q, eg, pxf, txf, c2g)
    return _tc_epilogue(part)
